# baseline (speedup 1.0000x reference)
import jax
import jax.numpy as jnp
from jax.experimental import pallas as pl
from jax.experimental.pallas import tpu as pltpu


def _copy_dma(u_ref, ou_ref, sem):
    c = pltpu.make_async_copy(u_ref, ou_ref, sem)
    c.start()
    c.wait()


def kernel(unary, binary, index1, index2):
    out_u = pl.pallas_call(
        _copy_dma,
        in_specs=[pl.BlockSpec(memory_space=pl.ANY)],
        out_specs=pl.BlockSpec(memory_space=pl.ANY),
        out_shape=jax.ShapeDtypeStruct(unary.shape, unary.dtype),
        scratch_shapes=[pltpu.SemaphoreType.DMA],
    )(unary)
    return out_u, binary


# speedup vs baseline: 1.8531x; 1.8531x over previous
import jax
import jax.numpy as jnp
from jax.experimental import pallas as pl
from jax.experimental.pallas import tpu as pltpu


def _copy_kernel(u_ref, b_ref, ou_ref, ob_ref, su, sb):
    cu = pltpu.make_async_copy(u_ref, ou_ref, su)
    cb = pltpu.make_async_copy(b_ref, ob_ref, sb)
    cu.start()
    cb.start()
    cu.wait()
    cb.wait()


def kernel(unary, binary, index1, index2):
    uT = unary.T
    bT = binary.T
    ouT, obT = pl.pallas_call(
        _copy_kernel,
        in_specs=[
            pl.BlockSpec(memory_space=pl.ANY),
            pl.BlockSpec(memory_space=pl.ANY),
        ],
        out_specs=[
            pl.BlockSpec(memory_space=pl.ANY),
            pl.BlockSpec(memory_space=pl.ANY),
        ],
        out_shape=[
            jax.ShapeDtypeStruct(uT.shape, uT.dtype),
            jax.ShapeDtypeStruct(bT.shape, bT.dtype),
        ],
        scratch_shapes=[pltpu.SemaphoreType.DMA, pltpu.SemaphoreType.DMA],
    )(uT, bT)
    return ouT.T, obT.T


# speedup vs baseline: 44.0478x; 23.7692x over previous
import jax
import jax.numpy as jnp
from jax.experimental import pallas as pl
from jax.experimental.pallas import tpu as pltpu

_B_LANES = 80000


def _copy2(u_ref, b_ref, ou_ref, ob_ref):
    i = pl.program_id(0)

    @pl.when(i == 0)
    def _():
        ou_ref[...] = u_ref[...]

    ob_ref[...] = b_ref[...]


def kernel(unary, binary, index1, index2):
    uT = unary.T
    bT = binary.T
    ouT, obT = pl.pallas_call(
        _copy2,
        grid=(1600000 // _B_LANES,),
        in_specs=[
            pl.BlockSpec((8, 50000), lambda i: (0, 0)),
            pl.BlockSpec((2, _B_LANES), lambda i: (0, i)),
        ],
        out_specs=[
            pl.BlockSpec((8, 50000), lambda i: (0, 0)),
            pl.BlockSpec((2, _B_LANES), lambda i: (0, i)),
        ],
        out_shape=[
            jax.ShapeDtypeStruct(uT.shape, uT.dtype),
            jax.ShapeDtypeStruct(bT.shape, bT.dtype),
        ],
    )(uT, bT)
    return ouT.T, obT.T
